# Initial kernel scaffold; baseline (speedup 1.0000x reference)
import jax
import jax.numpy as jnp
from jax.experimental import pallas as pl


def kernel(x, embeddings):
    raise NotImplementedError("write your pallas kernel here")


# speedup vs baseline: 1.0939x; 1.0939x over previous
import functools

import jax
import jax.numpy as jnp
from jax import lax
from jax.experimental import pallas as pl
from jax.experimental.pallas import tpu as pltpu
from jax.experimental.pallas import tpu_sc as plsc


_NC, _NS = 2, 16
_NW = _NC * _NS
_CHUNK = 1024


def kernel(x, embeddings):
    B, T = x.shape
    V, D = embeddings.shape
    N = B * T
    S = N // _NW
    n_chunks = S // _CHUNK
    idx = x.reshape(N)

    mesh = plsc.VectorSubcoreMesh(core_axis_name="c", subcore_axis_name="s")

    @functools.partial(
        pl.kernel,
        mesh=mesh,
        out_type=jax.ShapeDtypeStruct((N, D), embeddings.dtype),
        scratch_types=[
            pltpu.VMEM((_CHUNK,), jnp.int32),
            pltpu.VMEM((_CHUNK, D), jnp.float32),
            pltpu.SemaphoreType.DMA,
        ],
        compiler_params=pltpu.CompilerParams(use_tc_tiling_on_sc=False),
    )
    def gather_kernel(table_hbm, idx_hbm, out_hbm, idx_v, rows_v, sem):
        wid = lax.axis_index("s") * _NC + lax.axis_index("c")
        base = wid * S

        @pl.loop(0, n_chunks)
        def _(g):
            off = base + g * _CHUNK
            pltpu.sync_copy(idx_hbm.at[pl.ds(off, _CHUNK)], idx_v)
            pltpu.async_copy(table_hbm.at[idx_v], rows_v, sem).wait()
            pltpu.sync_copy(rows_v, out_hbm.at[pl.ds(off, _CHUNK)])

    out = gather_kernel(embeddings, idx)
    return out.reshape(B, T, D)


# speedup vs baseline: 1.4539x; 1.3290x over previous
import functools

import jax
import jax.numpy as jnp
from jax import lax
from jax.experimental import pallas as pl
from jax.experimental.pallas import tpu as pltpu
from jax.experimental.pallas import tpu_sc as plsc


_NC, _NS = 2, 16
_NW = _NC * _NS
_TT = 7
_BC = 128
_ITEMS = _TT * _BC
_PER_W = _ITEMS // _NW
_CHUNK = 1024


def kernel(x, embeddings):
    B, T = x.shape
    V, D = embeddings.shape

    xT = x.T
    fill = ((jnp.arange(6 * B, dtype=jnp.int32) * 7919) % V).reshape(6, B)
    xTp = jnp.concatenate([xT, fill], axis=0)
    idxR = xTp.reshape(_TT, 8, _BC, 128).transpose(0, 2, 1, 3)
    idx4 = (idxR * 4).reshape(-1)

    tableP = jnp.pad(embeddings, ((0, 0), (0, 128 - D))).reshape(4 * V, D)

    mesh = plsc.VectorSubcoreMesh(core_axis_name="c", subcore_axis_name="s")

    @functools.partial(
        pl.kernel,
        mesh=mesh,
        out_type=jax.ShapeDtypeStruct((T, 4, _BC, 8, 128), jnp.float32),
        scratch_types=[
            pltpu.VMEM((_PER_W * _CHUNK,), jnp.int32),
            pltpu.VMEM((2, _CHUNK, D), jnp.float32),
            pltpu.VMEM((D, 128), jnp.float32),
            pltpu.SemaphoreType.DMA,
        ],
        compiler_params=pltpu.CompilerParams(
            use_tc_tiling_on_sc=False, needs_layout_passes=False),
    )
    def gather_kernel(table_hbm, idx_hbm, out_hbm, idx_v, rows_v, tbuf, sem):
        wid = lax.axis_index("s") * _NC + lax.axis_index("c")
        item0 = wid * _PER_W

        pltpu.sync_copy(
            idx_hbm.at[pl.ds(item0 * _CHUNK, _PER_W * _CHUNK)], idx_v)

        def start_gather(g, buf):
            pltpu.async_copy(
                table_hbm.at[idx_v.at[pl.ds(g * _CHUNK, _CHUNK)]],
                rows_v.at[buf], sem)

        def wait_gather(buf):
            pltpu.make_async_copy(
                table_hbm.at[pl.ds(0, _CHUNK)], rows_v.at[buf], sem).wait()

        iota16 = lax.broadcasted_iota(jnp.int32, (16,), 0)

        def process(g, buf):
            item = item0 + g
            tt = item // _BC
            c = item % _BC
            for s_t in range(8):
                t = tt * 8 + s_t

                @pl.when(t < T)
                def _():
                    @pl.loop(0, D)
                    def _(d):
                        dcol = jnp.broadcast_to(d, (16,))
                        for grp in range(8):
                            ridx = iota16 + (s_t * 128 + grp * 16)
                            vals = plsc.load_gather(
                                rows_v.at[buf], [ridx, dcol])
                            tbuf[d, pl.ds(grp * 16, 16)] = vals
                    for r in range(4):
                        pltpu.sync_copy(
                            tbuf.at[pl.ds(r * 8, 8)],
                            out_hbm.at[t, r, c])

        start_gather(0, 0)

        @pl.loop(0, _PER_W, step=2)
        def _(g0):
            for b in range(2):
                g = g0 + b

                @pl.when(g + 1 < _PER_W)
                def _():
                    start_gather(g + 1, (b + 1) % 2)

                wait_gather(b)
                process(g, b)

    out5 = gather_kernel(tableP, idx4)
    return out5.transpose(2, 4, 0, 1, 3).reshape(B, T, D)
